# Initial kernel scaffold; baseline (speedup 1.0000x reference)
import jax
import jax.numpy as jnp
from jax.experimental import pallas as pl


def kernel(x, pos, edge_index, edge_attr_in, edge_attr_inh, pe, batch, params):
    raise NotImplementedError("write your pallas kernel here")


# speedup vs baseline: 5.8451x; 5.8451x over previous
import functools

import jax
import jax.numpy as jnp
import numpy as np
from jax.experimental import pallas as pl
from jax.experimental.pallas import tpu as pltpu

HID = 64
NRAD = 8
RMAX = 2.5


def _silu(x):
    return x * jax.lax.logistic(x)


def _sph(vec):
    n = jnp.linalg.norm(vec, axis=-1, keepdims=True)
    u = vec / jnp.maximum(n, 1e-12)
    x, y, z = u[..., 0], u[..., 1], u[..., 2]
    l0 = jnp.ones_like(x)
    s3 = np.sqrt(3.0)
    l1 = jnp.stack([s3 * x, s3 * y, s3 * z], axis=-1)
    s15 = np.sqrt(15.0)
    s5 = np.sqrt(5.0)
    l2 = jnp.stack([s15 * x * y, s15 * y * z, 0.5 * s5 * (3.0 * z * z - 1.0),
                    s15 * x * z, 0.5 * s15 * (x * x - y * y)], axis=-1)
    return jnp.concatenate([l0[..., None], l1, l2], axis=-1)


def _bessel(d, number, end):
    x = d[..., None]
    nvec = jnp.arange(1, number + 1, dtype=jnp.float32)
    safe = jnp.maximum(x, 1e-9)
    out = jnp.sqrt(2.0 / end) * jnp.sin(nvec * jnp.pi * x / end) / safe
    out = out * (x < end).astype(jnp.float32) * (x > 0).astype(jnp.float32)
    return out


def _edge_block(E):
    for b in (8000, 4000, 2000, 1000, 800, 400, 200, 100, 64, 32, 16, 8):
        if E % b == 0:
            return b
    return E


def _edge_mlp_body(msrc_ref, mdst_ref, rbf_ref, inv_ref,
                   wrbf_ref, brbf_ref, winv_ref, wmid_ref, wout_ref, wgate_ref,
                   e_ref, gate_ref):
    a = msrc_ref[...] + mdst_ref[...]
    a = a + jnp.dot(rbf_ref[...], wrbf_ref[...], preferred_element_type=jnp.float32)
    a = a + brbf_ref[...]
    a = a + jnp.dot(inv_ref[...], winv_ref[...], preferred_element_type=jnp.float32)
    e1 = _silu(a)
    e2 = _silu(jnp.dot(e1, wmid_ref[...], preferred_element_type=jnp.float32))
    e = jnp.dot(e2, wout_ref[...], preferred_element_type=jnp.float32)
    e_ref[...] = e
    gate_ref[...] = jnp.dot(e, wgate_ref[...], preferred_element_type=jnp.float32)


def _edge_mlp(msrc, mdst, rbf, inv, cp):
    E = msrc.shape[0]
    B = _edge_block(E)
    grid = (E // B,)
    row = lambda i: (i, 0)
    full = lambda i: (0, 0)
    e, gate = pl.pallas_call(
        _edge_mlp_body,
        grid=grid,
        in_specs=[
            pl.BlockSpec((B, HID), row),
            pl.BlockSpec((B, HID), row),
            pl.BlockSpec((B, rbf.shape[1]), row),
            pl.BlockSpec((B, 3), row),
            pl.BlockSpec(cp['W_rbf'].shape, full),
            pl.BlockSpec((1, HID), full),
            pl.BlockSpec((3, HID), full),
            pl.BlockSpec((HID, HID), full),
            pl.BlockSpec((HID, HID), full),
            pl.BlockSpec((HID, 1), full),
        ],
        out_specs=[
            pl.BlockSpec((B, HID), row),
            pl.BlockSpec((B, 1), row),
        ],
        out_shape=[
            jax.ShapeDtypeStruct((E, HID), jnp.float32),
            jax.ShapeDtypeStruct((E, 1), jnp.float32),
        ],
    )(msrc, mdst, rbf, inv, cp['W_rbf'], cp['b_rbf'][None, :], cp['W_inv'],
      cp['W_mid'], cp['W_out'], cp['W_gate'])
    return e, gate


def kernel(x, pos, edge_index, edge_attr_in, edge_attr_inh, pe, batch, params):
    N = pos.shape[0]
    src = edge_index[0]
    dst = edge_index[1]
    x_oh = jax.nn.one_hot(x - 1, 2, dtype=jnp.float32)
    norm_in = jnp.linalg.norm(edge_attr_in, axis=1)
    norm_inh = jnp.linalg.norm(edge_attr_inh, axis=1)
    rbf2 = _bessel(norm_in, NRAD, RMAX)
    rbf1 = _bessel(norm_inh, NRAD, RMAX)
    edge_sh = _sph(edge_attr_in)
    p = _sph(pos)
    rbf = jnp.concatenate([(norm_inh - norm_in)[:, None], rbf1, rbf2], axis=-1)
    feat = jnp.concatenate([x_oh, pe[:, None],
                            jnp.linalg.norm(pos, axis=-1, keepdims=True)], axis=-1)
    h = feat @ params['emb_W'] + params['emb_b']
    m = h
    E = src.shape[0]
    deg = jax.ops.segment_sum(jnp.ones((E,), jnp.float32), dst, num_segments=N)
    deg = jnp.maximum(deg, 1.0)[:, None]
    for cp in params['convs']:
        ms = m @ cp['W_src']
        md = m @ cp['W_dst']
        psrc = p[src]
        inv = jnp.stack([
            (psrc[:, 0:1] * edge_sh[:, 0:1]).sum(-1),
            (psrc[:, 1:4] * edge_sh[:, 1:4]).sum(-1),
            (psrc[:, 4:9] * edge_sh[:, 4:9]).sum(-1)], axis=-1)
        e, gate = _edge_mlp(ms[src], md[dst], rbf, inv, cp)
        m = m + jax.ops.segment_sum(e, dst, num_segments=N) / deg
        p = p + jax.ops.segment_sum(edge_sh * gate, dst, num_segments=N) / deg
    x2 = _silu(jnp.concatenate([h, m], axis=-1) @ params['cat_W'] + params['cat_b'])
    a1 = x2 @ params['agg_W'][:HID]
    a2 = x2 @ params['agg_W'][HID:]
    ef = a1[src] + a2[dst] + params['agg_b']
    nagg = jax.ops.segment_sum(ef, dst, num_segments=N) / deg
    out = jax.nn.relu(nagg @ params['out_W1']) @ params['out_W2']
    return (out, m)


# speedup vs baseline: 10.4327x; 1.7849x over previous
import functools

import jax
import jax.numpy as jnp
import numpy as np
from jax import lax
from jax.experimental import pallas as pl
from jax.experimental.pallas import tpu as pltpu
from jax.experimental.pallas import tpu_sc as plsc

HID = 64
NRAD = 8
RMAX = 2.5
PW = 16

NC = 2
NS = 16
NW = NC * NS
LANE = 16
CE = 128
CF = 64
NDUM = 128
WB = 1000


def _silu(x):
    return x * jax.lax.logistic(x)


def _sph(vec):
    n = jnp.linalg.norm(vec, axis=-1, keepdims=True)
    u = vec / jnp.maximum(n, 1e-12)
    x, y, z = u[..., 0], u[..., 1], u[..., 2]
    l0 = jnp.ones_like(x)
    s3 = np.sqrt(3.0)
    l1 = jnp.stack([s3 * x, s3 * y, s3 * z], axis=-1)
    s15 = np.sqrt(15.0)
    s5 = np.sqrt(5.0)
    l2 = jnp.stack([s15 * x * y, s15 * y * z, 0.5 * s5 * (3.0 * z * z - 1.0),
                    s15 * x * z, 0.5 * s15 * (x * x - y * y)], axis=-1)
    return jnp.concatenate([l0[..., None], l1, l2], axis=-1)


def _bessel(d, number, end):
    x = d[..., None]
    nvec = jnp.arange(1, number + 1, dtype=jnp.float32)
    safe = jnp.maximum(x, 1e-9)
    out = jnp.sqrt(2.0 / end) * jnp.sin(nvec * jnp.pi * x / end) / safe
    out = out * (x < end).astype(jnp.float32) * (x > 0).astype(jnp.float32)
    return out


def _blk(n, prefs):
    for b in prefs:
        if n % b == 0:
            return b
    return n


def _sc_mesh():
    return plsc.VectorSubcoreMesh(core_axis_name="c", subcore_axis_name="s")


def _sc_gather(t1, t2, src_g, dst_g):
    Ep = src_g.shape[0]
    W = t1.shape[1]
    nch = Ep // (CE * NW)

    @functools.partial(
        pl.kernel,
        out_type=[jax.ShapeDtypeStruct((Ep, W), jnp.float32),
                  jax.ShapeDtypeStruct((Ep, W), jnp.float32)],
        mesh=_sc_mesh(),
        scratch_types=[pltpu.VMEM((CE,), jnp.int32),
                       pltpu.VMEM((CE,), jnp.int32),
                       pltpu.VMEM((CE, W), jnp.float32),
                       pltpu.VMEM((CE, W), jnp.float32),
                       pltpu.SemaphoreType.DMA,
                       pltpu.SemaphoreType.DMA],
    )
    def k(t1_h, t2_h, src_h, dst_h, esrc_o, edst_o,
          sbuf, dbuf, b1, b2, sem1, sem2):
        wid = lax.axis_index("s") * NC + lax.axis_index("c")

        def body(j, carry):
            base = (wid * nch + j) * CE
            pltpu.sync_copy(src_h.at[pl.ds(base, CE)], sbuf)
            pltpu.sync_copy(dst_h.at[pl.ds(base, CE)], dbuf)
            c1 = pltpu.async_copy(t1_h.at[sbuf], b1, sem1)
            c2 = pltpu.async_copy(t2_h.at[dbuf], b2, sem2)
            c1.wait()
            c2.wait()
            pltpu.sync_copy(b1, esrc_o.at[pl.ds(base, CE)])
            pltpu.sync_copy(b2, edst_o.at[pl.ds(base, CE)])
            return carry

        lax.fori_loop(0, nch, body, 0)

    return k(t1, t2, src_g, dst_g)


def _sc_scatter_m(e, dst_s, z64, N):
    Ep = dst_s.shape[0]
    half = N // 2
    rows = half + NDUM
    nch = Ep // (CE * NS)
    nwb = half // WB

    @functools.partial(
        pl.kernel,
        out_type=jax.ShapeDtypeStruct((N, HID), jnp.float32),
        mesh=_sc_mesh(),
        scratch_types=[pltpu.VMEM((CE,), jnp.int32),
                       pltpu.VMEM((CE,), jnp.int32),
                       pltpu.VMEM((CE, HID), jnp.float32),
                       pltpu.VMEM_SHARED((rows, HID), jnp.float32)],
    )
    def k(e_h, dst_h, z64_h, magg_o, dbuf, ibuf, ebuf, mtab):
        cid = lax.axis_index("c")
        sid = lax.axis_index("s")
        lo = cid * half

        for j in range(2):
            wb = sid + j * NS
            @pl.when(wb < nwb)
            def _():
                off = wb * WB
                pltpu.sync_copy(z64_h.at[pl.ds(0, WB)], mtab.at[pl.ds(off, WB)])
        plsc.subcore_barrier()

        def body(j, carry):
            base = (sid * nch + j) * CE
            pltpu.sync_copy(dst_h.at[pl.ds(base, CE)], dbuf)

            def idx_body(kk, c2):
                d = dbuf[pl.ds(kk * LANE, LANE)]
                inb = (d >= lo) & (d < lo + half)
                loc = jnp.where(inb, d - lo, half + (d & (NDUM - 1)))
                ibuf[pl.ds(kk * LANE, LANE)] = loc
                return c2
            lax.fori_loop(0, CE // LANE, idx_body, 0)

            pltpu.sync_copy(e_h.at[pl.ds(base, CE)], ebuf)
            pltpu.sync_copy(ebuf, mtab.at[ibuf], add=True)
            return carry

        lax.fori_loop(0, nch, body, 0)
        plsc.subcore_barrier()

        for j in range(2):
            wb = sid + j * NS
            @pl.when(wb < nwb)
            def _():
                off = wb * WB
                pltpu.sync_copy(mtab.at[pl.ds(off, WB)],
                                magg_o.at[pl.ds(lo + off, WB)])

    return k(e, dst_s, z64)


def _sc_scatter_pd(gsh, dst_s, z16, N):
    Ep = dst_s.shape[0]
    half = N // 2
    rows = half + NDUM
    nch = Ep // (CE * NS)
    nwb = half // WB

    @functools.partial(
        pl.kernel,
        out_type=jax.ShapeDtypeStruct((N, PW), jnp.float32),
        mesh=_sc_mesh(),
        scratch_types=[pltpu.VMEM((CE,), jnp.int32),
                       pltpu.VMEM((CE,), jnp.int32),
                       pltpu.VMEM((CE, PW), jnp.float32),
                       pltpu.VMEM_SHARED((rows, PW), jnp.float32)],
    )
    def k(gsh_h, dst_h, z16_h, pagg_o, dbuf, ibuf, gbuf, ptab):
        cid = lax.axis_index("c")
        sid = lax.axis_index("s")
        lo = cid * half

        for j in range(2):
            wb = sid + j * NS
            @pl.when(wb < nwb)
            def _():
                off = wb * WB
                pltpu.sync_copy(z16_h.at[pl.ds(0, WB)], ptab.at[pl.ds(off, WB)])
        plsc.subcore_barrier()

        def body(j, carry):
            base = (sid * nch + j) * CE
            pltpu.sync_copy(dst_h.at[pl.ds(base, CE)], dbuf)

            def idx_body(kk, c2):
                d = dbuf[pl.ds(kk * LANE, LANE)]
                inb = (d >= lo) & (d < lo + half)
                loc = jnp.where(inb, d - lo, half + (d & (NDUM - 1)))
                ibuf[pl.ds(kk * LANE, LANE)] = loc
                return c2
            lax.fori_loop(0, CE // LANE, idx_body, 0)

            pltpu.sync_copy(gsh_h.at[pl.ds(base, CE)], gbuf)
            pltpu.sync_copy(gbuf, ptab.at[ibuf], add=True)
            return carry

        lax.fori_loop(0, nch, body, 0)
        plsc.subcore_barrier()

        for j in range(2):
            wb = sid + j * NS
            @pl.when(wb < nwb)
            def _():
                off = wb * WB
                pltpu.sync_copy(ptab.at[pl.ds(off, WB)], pagg_o.at[pl.ds(lo + off, WB)])

    return k(gsh, dst_s, z16)


def _sc_gather_scatter(a1t, src_g, dst_s, z128):
    Ep = src_g.shape[0]
    N = a1t.shape[0]
    qp = ((N // 4 + WB - 1) // WB) * WB
    rows = qp + NDUM
    nch = Ep // (CE * NS)
    nwb = qp // WB
    H = CE // 2

    @functools.partial(
        pl.kernel,
        out_type=jax.ShapeDtypeStruct((4 * qp, 128), jnp.float32),
        mesh=_sc_mesh(),
        scratch_types=[pltpu.VMEM((CE,), jnp.int32),
                       pltpu.VMEM((CE,), jnp.int32),
                       pltpu.VMEM((H,), jnp.int32),
                       pltpu.VMEM((H,), jnp.int32),
                       pltpu.VMEM((H,), jnp.int32),
                       pltpu.VMEM((H,), jnp.int32),
                       pltpu.VMEM((H, 128), jnp.float32),
                       pltpu.VMEM_SHARED((rows, 128), jnp.float32),
                       pltpu.SemaphoreType.DMA],
    )
    def k(a1_h, src_h, dst_h, z128_h, sagg_o,
          sbuf, dbuf, s0, s1, i0, i1, abuf, mtab, sem):
        cid = lax.axis_index("c")
        sid = lax.axis_index("s")

        for cpass in range(2):
            lo = (cpass * NC + cid) * qp
            for j in range(2):
                wb = sid + j * NS
                @pl.when(wb < nwb)
                def _():
                    pltpu.sync_copy(z128_h.at[pl.ds(0, WB)],
                                    mtab.at[pl.ds(wb * WB, WB)])
            plsc.subcore_barrier()

            def body(j, carry):
                base = (sid * nch + j) * CE
                pltpu.sync_copy(src_h.at[pl.ds(base, CE)], sbuf)
                pltpu.sync_copy(dst_h.at[pl.ds(base, CE)], dbuf)

                def idx_body(kk, c2):
                    d = dbuf[pl.ds(kk * LANE, LANE)]
                    inb = (d >= lo) & (d < lo + qp)
                    loc = jnp.where(inb, d - lo, qp + (d & (NDUM - 1)))
                    s = sbuf[pl.ds(kk * LANE, LANE)]
                    hh = kk // (H // LANE)
                    off = (kk % (H // LANE)) * LANE
                    (i0, i1)[hh][pl.ds(off, LANE)] = loc
                    (s0, s1)[hh][pl.ds(off, LANE)] = s
                    return c2
                for kk in range(CE // LANE):
                    idx_body(kk, 0)

                for hh in range(2):
                    pltpu.async_copy(a1_h.at[(s0, s1)[hh]], abuf, sem).wait()
                    pltpu.sync_copy(abuf, mtab.at[(i0, i1)[hh]], add=True)
                return carry

            lax.fori_loop(0, nch, body, 0)
            plsc.subcore_barrier()

            for j in range(2):
                wb = sid + j * NS
                @pl.when(wb < nwb)
                def _():
                    off = wb * WB
                    pltpu.sync_copy(mtab.at[pl.ds(off, WB)],
                                    sagg_o.at[pl.ds(lo + off, WB)])
            plsc.subcore_barrier()

    return k(a1t, src_g, dst_s, z128)


def _node_body(m_ref, aggm_ref, deg_ref, p_ref, aggp_ref, wsrc_ref, wdst_ref,
               t1_ref, t2_ref, mnew_ref, pnew_ref):
    m_new = m_ref[...] + aggm_ref[...] / deg_ref[...]
    p_new = p_ref[...] + aggp_ref[...] / deg_ref[...]
    B = m_new.shape[0]
    ms = jnp.dot(m_new, wsrc_ref[...], preferred_element_type=jnp.float32)
    md = jnp.dot(m_new, wdst_ref[...], preferred_element_type=jnp.float32)
    z = jnp.zeros((B, 128 - HID - PW), jnp.float32)
    t1_ref[...] = jnp.concatenate([ms, p_new, z], axis=1)
    t2_ref[...] = jnp.concatenate([md, p_new, z], axis=1)
    mnew_ref[...] = m_new
    pnew_ref[...] = p_new


def _node_transform(m, aggm, deg, p16, aggp, cp):
    N = m.shape[0]
    B = _blk(N, (5000, 2000, 1000, 400, 200, 40, 8))
    row = lambda i: (i, 0)
    full = lambda i: (0, 0)
    return pl.pallas_call(
        _node_body,
        grid=(N // B,),
        in_specs=[pl.BlockSpec((B, HID), row), pl.BlockSpec((B, HID), row),
                  pl.BlockSpec((B, 1), row), pl.BlockSpec((B, PW), row),
                  pl.BlockSpec((B, PW), row),
                  pl.BlockSpec((HID, HID), full), pl.BlockSpec((HID, HID), full)],
        out_specs=[pl.BlockSpec((B, 128), row), pl.BlockSpec((B, 128), row),
                   pl.BlockSpec((B, HID), row), pl.BlockSpec((B, PW), row)],
        out_shape=[jax.ShapeDtypeStruct((N, 128), jnp.float32),
                   jax.ShapeDtypeStruct((N, 128), jnp.float32),
                   jax.ShapeDtypeStruct((N, HID), jnp.float32),
                   jax.ShapeDtypeStruct((N, PW), jnp.float32)],
    )(m, aggm, deg, p16, aggp, cp['W_src'], cp['W_dst'])


def _edge_body(esrc_ref, edst_ref, rbf_ref, sh_ref,
               wrbf_ref, brbf_ref, winv_ref, wmid_ref, wout_ref, wgate_ref,
               e_ref, gsh_ref):
    msrc = esrc_ref[:, :HID]
    psrc = esrc_ref[:, HID:HID + PW]
    a = msrc + edst_ref[:, :HID]
    a = a + jnp.dot(rbf_ref[...], wrbf_ref[...], preferred_element_type=jnp.float32)
    a = a + brbf_ref[...]
    a = a + jnp.dot(psrc * sh_ref[...], winv_ref[...],
                    preferred_element_type=jnp.float32)
    e1 = _silu(a)
    e2 = _silu(jnp.dot(e1, wmid_ref[...], preferred_element_type=jnp.float32))
    e = jnp.dot(e2, wout_ref[...], preferred_element_type=jnp.float32)
    e_ref[...] = e
    gate = jnp.dot(e, wgate_ref[...], preferred_element_type=jnp.float32)
    g = sh_ref[...] * gate
    col = lax.broadcasted_iota(jnp.int32, g.shape, 1)
    gsh_ref[...] = jnp.where(col == 9, 1.0, g)


def _edge_mlp(esrc, edst, rbf, sh16, cp, winv16):
    Ep = esrc.shape[0]
    B = _blk(Ep, (8192, 4096, 2048, 1024, 512, 256, 128, 64, 32, 16, 8))
    grid = (Ep // B,)
    row = lambda i: (i, 0)
    full = lambda i: (0, 0)
    R = rbf.shape[1]
    return pl.pallas_call(
        _edge_body,
        grid=grid,
        in_specs=[pl.BlockSpec((B, 128), row), pl.BlockSpec((B, 128), row),
                  pl.BlockSpec((B, R), row), pl.BlockSpec((B, PW), row),
                  pl.BlockSpec((R, HID), full), pl.BlockSpec((1, HID), full),
                  pl.BlockSpec((PW, HID), full), pl.BlockSpec((HID, HID), full),
                  pl.BlockSpec((HID, HID), full), pl.BlockSpec((HID, 1), full)],
        out_specs=[pl.BlockSpec((B, HID), row), pl.BlockSpec((B, PW), row)],
        out_shape=[jax.ShapeDtypeStruct((Ep, HID), jnp.float32),
                   jax.ShapeDtypeStruct((Ep, PW), jnp.float32)],
    )(esrc, edst, rbf, sh16, cp['W_rbf'], cp['b_rbf'][None, :], winv16,
      cp['W_mid'], cp['W_out'], cp['W_gate'])


def _final_node_body(h_ref, m_ref, aggm_ref, deg_ref, catw_ref, catb_ref,
                     wa_ref, wb_ref, mnew_ref, a1_ref, a2_ref):
    m_new = m_ref[...] + aggm_ref[...] / deg_ref[...]
    x2 = _silu(jnp.dot(jnp.concatenate([h_ref[...], m_new], axis=1), catw_ref[...],
                       preferred_element_type=jnp.float32) + catb_ref[...])
    mnew_ref[...] = m_new
    a1 = jnp.dot(x2, wa_ref[...], preferred_element_type=jnp.float32)
    a1_ref[...] = jnp.concatenate([a1, jnp.zeros_like(a1)], axis=1)
    a2_ref[...] = jnp.dot(x2, wb_ref[...], preferred_element_type=jnp.float32)


def _final_node(h, m, aggm, deg, params):
    N = m.shape[0]
    B = _blk(N, (5000, 2000, 1000, 400, 200, 40, 8))
    row = lambda i: (i, 0)
    full = lambda i: (0, 0)
    return pl.pallas_call(
        _final_node_body,
        grid=(N // B,),
        in_specs=[pl.BlockSpec((B, HID), row), pl.BlockSpec((B, HID), row),
                  pl.BlockSpec((B, HID), row), pl.BlockSpec((B, 1), row),
                  pl.BlockSpec((2 * HID, HID), full), pl.BlockSpec((1, HID), full),
                  pl.BlockSpec((HID, HID), full), pl.BlockSpec((HID, HID), full)],
        out_specs=[pl.BlockSpec((B, HID), row), pl.BlockSpec((B, 128), row),
                   pl.BlockSpec((B, HID), row)],
        out_shape=[jax.ShapeDtypeStruct((N, HID), jnp.float32),
                   jax.ShapeDtypeStruct((N, 128), jnp.float32),
                   jax.ShapeDtypeStruct((N, HID), jnp.float32)],
    )(h, m, aggm, deg, params['cat_W'], params['cat_b'][None, :],
      params['agg_W'][:HID], params['agg_W'][HID:])


_SC_GATHER = True
_SC_SCATTER = False
_SC_FUSED = False


def _xla_scatter_m(e, dst_s, z64, N):
    return jax.ops.segment_sum(e, dst_s, num_segments=N + 1)[:N]


def _xla_scatter_pd(gsh, dst_s, z16, N):
    return jax.ops.segment_sum(gsh, dst_s, num_segments=N + 1)[:N]


def _xla_gather(t1, t2, src_g, dst_g):
    return t1[src_g], t2[dst_g]


def _xla_gs(a1t, src_g, dst_s, z128):
    qp = a1t.shape[0] // 4
    return jax.ops.segment_sum(a1t[src_g], dst_s, num_segments=4 * qp)


def kernel(x, pos, edge_index, edge_attr_in, edge_attr_inh, pe, batch, params):
    N = pos.shape[0]
    E = edge_index.shape[1]
    half = N // 2
    assert N % 4 == 0 and half % WB == 0

    grain = CE * NW
    Ep = ((E + grain - 1) // grain) * grain
    pad = Ep - E

    src = edge_index[0].astype(jnp.int32)
    dst = edge_index[1].astype(jnp.int32)
    src_g = jnp.concatenate([src, jnp.zeros((pad,), jnp.int32)])
    dst_g = jnp.concatenate([dst, jnp.zeros((pad,), jnp.int32)])
    dst_s = jnp.concatenate([dst, jnp.full((pad,), N, jnp.int32)])

    eai = jnp.concatenate([edge_attr_in, jnp.zeros((pad, 3), jnp.float32)])
    eah = jnp.concatenate([edge_attr_inh, jnp.zeros((pad, 3), jnp.float32)])

    x_oh = jax.nn.one_hot(x - 1, 2, dtype=jnp.float32)
    norm_in = jnp.linalg.norm(eai, axis=1)
    norm_inh = jnp.linalg.norm(eah, axis=1)
    rbf2 = _bessel(norm_in, NRAD, RMAX)
    rbf1 = _bessel(norm_inh, NRAD, RMAX)
    sh16 = jnp.pad(_sph(eai), ((0, 0), (0, PW - 9)))
    p16 = jnp.pad(_sph(pos), ((0, 0), (0, PW - 9)))
    rbf = jnp.concatenate([(norm_inh - norm_in)[:, None], rbf1, rbf2], axis=-1)
    feat = jnp.concatenate([x_oh, pe[:, None],
                            jnp.linalg.norm(pos, axis=-1, keepdims=True)], axis=-1)
    h = feat @ params['emb_W'] + params['emb_b']

    z64 = jnp.zeros((WB, HID), jnp.float32)
    z16 = jnp.zeros((WB, PW), jnp.float32)
    z128 = jnp.zeros((WB, 128), jnp.float32)

    m = h
    aggm = jnp.zeros((N, HID), jnp.float32)
    aggp = jnp.zeros((N, PW), jnp.float32)
    deg = jnp.ones((N, 1), jnp.float32)
    for li, cp in enumerate(params['convs']):
        winv16 = jnp.concatenate([
            cp['W_inv'][0:1],
            jnp.tile(cp['W_inv'][1:2], (3, 1)),
            jnp.tile(cp['W_inv'][2:3], (5, 1)),
            jnp.zeros((PW - 9, HID), jnp.float32)], axis=0)
        t1, t2, m, p16 = _node_transform(m, aggm, deg, p16, aggp, cp)
        gat = _sc_gather if _SC_GATHER else _xla_gather
        esrc, edst = gat(t1, t2, src_g, dst_g)
        e, gsh = _edge_mlp(esrc, edst, rbf, sh16, cp, winv16)
        scm = _sc_scatter_m if _SC_SCATTER else _xla_scatter_m
        scp = _sc_scatter_pd if _SC_SCATTER else _xla_scatter_pd
        aggm = scm(e, dst_s, z64, N)
        aggp = scp(gsh, dst_s, z16, N)
        if li == 0:
            cnt = aggp[:, 9]
            deg = jnp.maximum(cnt, 1.0)[:, None]

    m, a1, a2 = _final_node(h, m, aggm, deg, params)
    gsf = _sc_gather_scatter if _SC_FUSED else _xla_gs
    sagg = gsf(a1, src_g, dst_s, z128)[:N, :HID]
    factor = cnt[:, None] / deg
    nagg = sagg / deg + factor * (a2 + params['agg_b'][None, :])
    out = jax.nn.relu(nagg @ params['out_W1']) @ params['out_W2']
    return (out, m)
